# baseline (speedup 1.0000x reference)
import functools

import jax
import jax.numpy as jnp
from jax import lax
from jax.experimental import pallas as pl
from jax.experimental.pallas import tpu as pltpu
from jax.experimental.pallas import tpu_sc as plsc

N = 10000
E = 320000
D = 128
H = 256
HH = 128
NPAD = 10240
NS = 16
NC = 2
CHUNK = 128
NCH = 158
EPT = NCH * CHUNK
EPAD = NS * EPT
EPD = E // (NC * NS)
BM = 400
GRID = N // BM
RPT = NPAD // NS


def _mesh():
    return plsc.VectorSubcoreMesh(core_axis_name="c", subcore_axis_name="s")


@functools.cache
def _deg_kernel():
    return pl.kernel(
        _deg_body,
        out_type=jax.ShapeDtypeStruct((NC * NPAD,), jnp.float32),
        mesh=_mesh(),
        compiler_params=pltpu.CompilerParams(needs_layout_passes=False),
        scratch_types=[
            pltpu.VMEM((EPD,), jnp.int32),
            pltpu.VMEM((NPAD,), jnp.float32),
            pltpu.VMEM((RPT,), jnp.float32),
            pltpu.VMEM((RPT,), jnp.float32),
            pltpu.VMEM_SHARED((NS, NPAD), jnp.float32),
        ],
    )


def _deg_body(dst_hbm, out_hbm, dstb, acc, red, res, shared):
    c = lax.axis_index("c")
    s = lax.axis_index("s")
    tile = c * NS + s
    pltpu.sync_copy(dst_hbm.at[pl.ds(tile * EPD, EPD)], dstb)

    zero16 = jnp.zeros((16,), jnp.float32)
    ones16 = jnp.ones((16,), jnp.float32)

    def zbody(i, _):
        acc[pl.ds(i * 16, 16)] = zero16
        return 0
    lax.fori_loop(0, NPAD // 16, zbody, 0)

    def ebody(i, _):
        idx = dstb[pl.ds(i * 16, 16)]
        plsc.addupdate_scatter(acc, [idx], ones16)
        return 0
    lax.fori_loop(0, EPD // 16, ebody, 0)

    pltpu.sync_copy(acc, shared.at[s])
    plsc.subcore_barrier()

    pltpu.sync_copy(shared.at[0, pl.ds(s * RPT, RPT)], res)

    def rbody(r, _):
        pltpu.sync_copy(shared.at[r, pl.ds(s * RPT, RPT)], red)

        def abody(j, _):
            res[pl.ds(j * 16, 16)] = res[pl.ds(j * 16, 16)] + red[pl.ds(j * 16, 16)]
            return 0
        lax.fori_loop(0, RPT // 16, abody, 0)
        return 0
    lax.fori_loop(1, NS, rbody, 0)

    pltpu.sync_copy(res, out_hbm.at[pl.ds(c * NPAD + s * RPT, RPT)])


@functools.cache
def _agg_kernel():
    return pl.kernel(
        _agg_body,
        out_type=jax.ShapeDtypeStruct((NC * NPAD, HH), jnp.float32),
        mesh=_mesh(),
        compiler_params=pltpu.CompilerParams(needs_layout_passes=False),
        scratch_types=[
            pltpu.VMEM((CHUNK,), jnp.int32),
            pltpu.VMEM((CHUNK,), jnp.int32),
            pltpu.VMEM((CHUNK,), jnp.int32),
            pltpu.VMEM((CHUNK,), jnp.int32),
            pltpu.VMEM((CHUNK, HH), jnp.float32),
            pltpu.VMEM((CHUNK, HH), jnp.float32),
            pltpu.VMEM_SHARED((NPAD, HH), jnp.float32),
            pltpu.SemaphoreType.DMA,
            pltpu.SemaphoreType.DMA,
            pltpu.SemaphoreType.DMA,
            pltpu.SemaphoreType.DMA,
        ],
    )


def _agg_body(g_hbm, src2_hbm, dst_hbm, out_hbm, gidx0, gidx1, didx0, didx1,
              rows0, rows1, acc, semi0, semi1, semg, sems):
    c = lax.axis_index("c")
    s = lax.axis_index("s")
    gidx = (gidx0, gidx1)
    didx = (didx0, didx1)
    rows = (rows0, rows1)
    semi = (semi0, semi1)

    def sslice(k):
        return pl.ds(c * EPAD + s * EPT + k * CHUNK, CHUNK)

    def dslice(k):
        return pl.ds(s * EPT + k * CHUNK, CHUNK)

    def issue_idx(k, b):
        pltpu.async_copy(src2_hbm.at[sslice(k)], gidx[b], semi[b])
        pltpu.async_copy(dst_hbm.at[dslice(k)], didx[b], semi[b])

    def wait_idx(k, b):
        pltpu.make_async_copy(src2_hbm.at[sslice(k)], gidx[b], semi[b]).wait()
        pltpu.make_async_copy(dst_hbm.at[dslice(k)], didx[b], semi[b]).wait()

    zero16 = jnp.zeros((16,), jnp.float32)

    def zbody(i, _):
        rows0[i // 8, pl.ds((i % 8) * 16, 16)] = zero16
        return 0
    lax.fori_loop(0, CHUNK * HH // 16, zbody, 0)

    def zcopy(i, _):
        pltpu.sync_copy(rows0, acc.at[pl.ds(s * RPT + i * CHUNK, CHUNK)])
        return 0
    lax.fori_loop(0, RPT // CHUNK, zcopy, 0)
    plsc.subcore_barrier()

    issue_idx(0, 0)

    def pair_body(i, _):
        for b in (0, 1):
            k = 2 * i + b
            ob = 1 - b
            wait_idx(k, b)
            gd = pltpu.async_copy(g_hbm.at[gidx[b]], rows[b], semg)

            @pl.when(k >= 1)
            def _wait_prev_scatter():
                pltpu.make_async_copy(rows[ob], acc.at[didx[ob]], sems).wait()

            @pl.when(k + 1 < NCH)
            def _prefetch_next_idx():
                issue_idx(k + 1, ob)

            gd.wait()
            pltpu.async_copy(rows[b], acc.at[didx[b]], sems, add=True)
        return 0
    lax.fori_loop(0, NCH // 2, pair_body, 0)
    pltpu.make_async_copy(rows[1], acc.at[didx[1]], sems).wait()
    plsc.subcore_barrier()

    def obody(i, _):
        r0 = s * RPT + i * CHUNK
        pltpu.sync_copy(acc.at[pl.ds(r0, CHUNK)], rows0)
        pltpu.sync_copy(rows0, out_hbm.at[pl.ds(c * NPAD + r0, CHUNK)])
        return 0
    lax.fori_loop(0, RPT // CHUNK, obody, 0)


@functools.cache
def _agg1_kernel():
    return pl.kernel(
        _agg1_body,
        out_type=jax.ShapeDtypeStruct((NC * NPAD,), jnp.float32),
        mesh=_mesh(),
        compiler_params=pltpu.CompilerParams(needs_layout_passes=False),
        scratch_types=[
            pltpu.VMEM((NPAD,), jnp.float32),
            pltpu.VMEM((EPD,), jnp.int32),
            pltpu.VMEM((EPD,), jnp.int32),
            pltpu.VMEM((NPAD,), jnp.float32),
            pltpu.VMEM((RPT,), jnp.float32),
            pltpu.VMEM((RPT,), jnp.float32),
            pltpu.VMEM_SHARED((NS, NPAD), jnp.float32),
        ],
    )


def _agg1_body(g3_hbm, src_hbm, dst_hbm, out_hbm, g3b, srcb, dstb, acc,
               red, res, shared):
    c = lax.axis_index("c")
    s = lax.axis_index("s")
    tile = c * NS + s
    pltpu.sync_copy(g3_hbm, g3b)
    pltpu.sync_copy(src_hbm.at[pl.ds(tile * EPD, EPD)], srcb)
    pltpu.sync_copy(dst_hbm.at[pl.ds(tile * EPD, EPD)], dstb)

    zero16 = jnp.zeros((16,), jnp.float32)

    def zbody(i, _):
        acc[pl.ds(i * 16, 16)] = zero16
        return 0
    lax.fori_loop(0, NPAD // 16, zbody, 0)

    def ebody(i, _):
        si = srcb[pl.ds(i * 16, 16)]
        di = dstb[pl.ds(i * 16, 16)]
        vals = plsc.load_gather(g3b, [si])
        plsc.addupdate_scatter(acc, [di], vals)
        return 0
    lax.fori_loop(0, EPD // 16, ebody, 0)

    pltpu.sync_copy(acc, shared.at[s])
    plsc.subcore_barrier()

    pltpu.sync_copy(shared.at[0, pl.ds(s * RPT, RPT)], res)

    def rbody(r, _):
        pltpu.sync_copy(shared.at[r, pl.ds(s * RPT, RPT)], red)

        def abody(j, _):
            res[pl.ds(j * 16, 16)] = res[pl.ds(j * 16, 16)] + red[pl.ds(j * 16, 16)]
            return 0
        lax.fori_loop(0, RPT // 16, abody, 0)
        return 0
    lax.fori_loop(1, NS, rbody, 0)

    pltpu.sync_copy(res, out_hbm.at[pl.ds(c * NPAD + s * RPT, RPT)])


def _dinv_of(degp_blk):
    deg = degp_blk[:, 0] + degp_blk[:, 1] + 1.0
    return lax.rsqrt(jnp.maximum(deg, 1.0))


def _tc1_body(x_ref, w1_ref, degp_ref, g1_ref):
    h = jnp.dot(x_ref[...].astype(jnp.bfloat16), w1_ref[...].astype(jnp.bfloat16),
                preferred_element_type=jnp.float32)
    dinv = _dinv_of(degp_ref[...])
    g = h * dinv[:, None]
    g1_ref[0] = g[:, :HH]
    g1_ref[1] = g[:, HH:]


def _tc2_body(agg_ref, g_ref, degp_ref, b_ref, w_ref, out_ref):
    dinv = _dinv_of(degp_ref[...])
    a = jnp.concatenate([agg_ref[0], agg_ref[1]], axis=1)
    g = jnp.concatenate([g_ref[0], g_ref[1]], axis=1)
    h = jnp.maximum(dinv[:, None] * (a + g) + b_ref[0, :], 0.0)
    g2 = jnp.dot(h.astype(jnp.bfloat16), w_ref[...].astype(jnp.bfloat16),
                 preferred_element_type=jnp.float32) * dinv[:, None]
    out_ref[0] = g2[:, :HH]
    out_ref[1] = g2[:, HH:]


def _tc3_body(agg_ref, g_ref, degp_ref, b_ref, w3_ref, out_ref):
    dinv = _dinv_of(degp_ref[...])
    a = jnp.concatenate([agg_ref[0], agg_ref[1]], axis=1)
    g = jnp.concatenate([g_ref[0], g_ref[1]], axis=1)
    h = jnp.maximum(dinv[:, None] * (a + g) + b_ref[0, :], 0.0)
    hb = h.astype(jnp.bfloat16).astype(jnp.float32)
    wb = w3_ref[0, :].astype(jnp.bfloat16).astype(jnp.float32)
    t = jnp.sum(hb * wb[None, :], axis=1) * dinv
    out_ref[...] = jnp.broadcast_to(t[:, None], (BM, 8))


def _tc4_body(p3_ref, g3_ref, degp_ref, b3_ref, out_ref):
    dinv = _dinv_of(degp_ref[...])
    agg = p3_ref[:, 0] + p3_ref[:, 1]
    res = dinv * (agg + g3_ref[:, 0]) + b3_ref[0, 0]
    out_ref[...] = jnp.broadcast_to(res[:, None], (BM, 8))


def _tc1(x, W1, degpT):
    return pl.pallas_call(
        _tc1_body,
        grid=(GRID,),
        in_specs=[
            pl.BlockSpec((BM, D), lambda i: (i, 0)),
            pl.BlockSpec((D, H), lambda i: (0, 0)),
            pl.BlockSpec((BM, 2), lambda i: (i, 0)),
        ],
        out_specs=pl.BlockSpec((2, BM, HH), lambda i: (0, i, 0)),
        out_shape=jax.ShapeDtypeStruct((2, N, HH), jnp.float32),
    )(x, W1, degpT)


def _tc2(agg, g, degpT, b, W):
    return pl.pallas_call(
        _tc2_body,
        grid=(GRID,),
        in_specs=[
            pl.BlockSpec((2, BM, HH), lambda i: (0, i, 0)),
            pl.BlockSpec((2, BM, HH), lambda i: (0, i, 0)),
            pl.BlockSpec((BM, 2), lambda i: (i, 0)),
            pl.BlockSpec((1, H), lambda i: (0, 0)),
            pl.BlockSpec((H, H), lambda i: (0, 0)),
        ],
        out_specs=pl.BlockSpec((2, BM, HH), lambda i: (0, i, 0)),
        out_shape=jax.ShapeDtypeStruct((2, N, HH), jnp.float32),
    )(agg, g, degpT, b, W)


def _tc3(agg, g, degpT, b, W3r):
    return pl.pallas_call(
        _tc3_body,
        grid=(GRID,),
        in_specs=[
            pl.BlockSpec((2, BM, HH), lambda i: (0, i, 0)),
            pl.BlockSpec((2, BM, HH), lambda i: (0, i, 0)),
            pl.BlockSpec((BM, 2), lambda i: (i, 0)),
            pl.BlockSpec((1, H), lambda i: (0, 0)),
            pl.BlockSpec((1, H), lambda i: (0, 0)),
        ],
        out_specs=pl.BlockSpec((BM, 8), lambda i: (i, 0)),
        out_shape=jax.ShapeDtypeStruct((N, 8), jnp.float32),
    )(agg, g, degpT, b, W3r)


def _tc4(p3T, g3r, degpT, b3):
    return pl.pallas_call(
        _tc4_body,
        grid=(GRID,),
        in_specs=[
            pl.BlockSpec((BM, 2), lambda i: (i, 0)),
            pl.BlockSpec((BM, 8), lambda i: (i, 0)),
            pl.BlockSpec((BM, 2), lambda i: (i, 0)),
            pl.BlockSpec((1, 1), lambda i: (0, 0)),
        ],
        out_specs=pl.BlockSpec((BM, 8), lambda i: (i, 0)),
        out_shape=jax.ShapeDtypeStruct((N, 8), jnp.float32),
    )(p3T, g3r, degpT, b3)


@jax.jit
def _run(x, src, dst, W1, b1, W2, b2, W3, b3):
    pad = EPAD - E
    srcp = jnp.concatenate([src, jnp.zeros((pad,), jnp.int32)])
    srcpp = jnp.concatenate([srcp, srcp + N])
    dstp = jnp.concatenate([dst, jnp.full((pad,), N, jnp.int32)])

    degpT = _deg_kernel()(dst).reshape(NC, NPAD).T
    g1 = _tc1(x, W1, degpT)
    agg1 = _agg_kernel()(g1.reshape(NC * N, HH), srcpp, dstp).reshape(NC, NPAD, HH)
    g2 = _tc2(agg1, g1, degpT, b1.reshape(1, H), W2)
    agg2 = _agg_kernel()(g2.reshape(NC * N, HH), srcpp, dstp).reshape(NC, NPAD, HH)
    g3 = _tc3(agg2, g2, degpT, b2.reshape(1, H), W3.reshape(1, H))
    g3p = jnp.concatenate([g3[:, 0], jnp.zeros((NPAD - N,), jnp.float32)])
    p3T = _agg1_kernel()(g3p, src, dst).reshape(NC, NPAD).T
    out = _tc4(p3T, g3, degpT, b3.reshape(1, 1))
    return out[:, :1]


def kernel(x, edge_index, edge_attr, W1, b1, W2, b2, W3, b3):
    src = edge_index[0].astype(jnp.int32)
    dst = edge_index[1].astype(jnp.int32)
    return _run(x, src, dst, W1, b1, W2, b2, W3, b3)


# speedup vs baseline: 1.4462x; 1.4462x over previous
import functools

import jax
import jax.numpy as jnp
from jax import lax
from jax.experimental import pallas as pl
from jax.experimental.pallas import tpu as pltpu
from jax.experimental.pallas import tpu_sc as plsc

N = 10000
E = 320000
D = 128
H = 256
HH = 128
NPAD = 10240
NS = 16
NC = 2
CHUNK = 88
NCH = 228
EPT = NCH * CHUNK
EPAD = NS * EPT
EPD = E // (NC * NS)
BM = 400
GRID = N // BM
RPT = NPAD // NS


def _mesh():
    return plsc.VectorSubcoreMesh(core_axis_name="c", subcore_axis_name="s")


@functools.cache
def _deg_kernel():
    return pl.kernel(
        _deg_body,
        out_type=jax.ShapeDtypeStruct((NC * NPAD,), jnp.float32),
        mesh=_mesh(),
        compiler_params=pltpu.CompilerParams(needs_layout_passes=False),
        scratch_types=[
            pltpu.VMEM((EPD,), jnp.int32),
            pltpu.VMEM((NPAD,), jnp.float32),
            pltpu.VMEM((RPT,), jnp.float32),
            pltpu.VMEM((RPT,), jnp.float32),
            pltpu.VMEM_SHARED((NS, NPAD), jnp.float32),
        ],
    )


def _deg_body(dst_hbm, out_hbm, dstb, acc, red, res, shared):
    c = lax.axis_index("c")
    s = lax.axis_index("s")
    tile = c * NS + s
    pltpu.sync_copy(dst_hbm.at[pl.ds(tile * EPD, EPD)], dstb)

    zero16 = jnp.zeros((16,), jnp.float32)
    ones16 = jnp.ones((16,), jnp.float32)

    def zbody(i, _):
        acc[pl.ds(i * 16, 16)] = zero16
        return 0
    lax.fori_loop(0, NPAD // 16, zbody, 0)

    def ebody(i, _):
        idx = dstb[pl.ds(i * 16, 16)]
        plsc.addupdate_scatter(acc, [idx], ones16)
        return 0
    lax.fori_loop(0, EPD // 16, ebody, 0)

    pltpu.sync_copy(acc, shared.at[s])
    plsc.subcore_barrier()

    pltpu.sync_copy(shared.at[0, pl.ds(s * RPT, RPT)], res)

    def rbody(r, _):
        pltpu.sync_copy(shared.at[r, pl.ds(s * RPT, RPT)], red)

        def abody(j, _):
            res[pl.ds(j * 16, 16)] = res[pl.ds(j * 16, 16)] + red[pl.ds(j * 16, 16)]
            return 0
        lax.fori_loop(0, RPT // 16, abody, 0)
        return 0
    lax.fori_loop(1, NS, rbody, 0)

    pltpu.sync_copy(res, out_hbm.at[pl.ds(c * NPAD + s * RPT, RPT)])


@functools.cache
def _agg_kernel():
    return pl.kernel(
        _agg_body,
        out_type=jax.ShapeDtypeStruct((NC * NPAD, HH), jnp.float32),
        mesh=_mesh(),
        compiler_params=pltpu.CompilerParams(needs_layout_passes=False),
        scratch_types=[
            pltpu.VMEM((CHUNK,), jnp.int32),
            pltpu.VMEM((CHUNK,), jnp.int32),
            pltpu.VMEM((CHUNK,), jnp.int32),
            pltpu.VMEM((CHUNK,), jnp.int32),
            pltpu.VMEM((CHUNK,), jnp.int32),
            pltpu.VMEM((CHUNK,), jnp.int32),
            pltpu.VMEM((CHUNK,), jnp.int32),
            pltpu.VMEM((CHUNK,), jnp.int32),
            pltpu.VMEM((CHUNK, HH), jnp.float32),
            pltpu.VMEM((CHUNK, HH), jnp.float32),
            pltpu.VMEM((CHUNK, HH), jnp.float32),
            pltpu.VMEM((CHUNK, HH), jnp.float32),
            pltpu.VMEM_SHARED((NPAD, HH), jnp.float32),
            pltpu.SemaphoreType.DMA,
            pltpu.SemaphoreType.DMA,
            pltpu.SemaphoreType.DMA,
            pltpu.SemaphoreType.DMA,
            pltpu.SemaphoreType.DMA,
            pltpu.SemaphoreType.DMA,
        ],
    )


def _agg_body(g_hbm, src2_hbm, dst_hbm, out_hbm,
              gidx0, gidx1, gidx2, gidx3, didx0, didx1, didx2, didx3,
              rows0, rows1, rows2, rows3, acc,
              semi0, semi1, semi2, semi3, semg, sems):
    c = lax.axis_index("c")
    s = lax.axis_index("s")
    gidx = (gidx0, gidx1, gidx2, gidx3)
    didx = (didx0, didx1, didx2, didx3)
    rows = (rows0, rows1, rows2, rows3)
    semi = (semi0, semi1, semi2, semi3)

    def sslice(k):
        return pl.ds(c * EPAD + s * EPT + k * CHUNK, CHUNK)

    def dslice(k):
        return pl.ds(s * EPT + k * CHUNK, CHUNK)

    def issue_idx(k, j):
        pltpu.async_copy(src2_hbm.at[sslice(k)], gidx[j], semi[j])
        pltpu.async_copy(dst_hbm.at[dslice(k)], didx[j], semi[j])

    def wait_idx(k, j):
        pltpu.make_async_copy(src2_hbm.at[sslice(k)], gidx[j], semi[j]).wait()
        pltpu.make_async_copy(dst_hbm.at[dslice(k)], didx[j], semi[j]).wait()

    def issue_gather(j):
        pltpu.async_copy(g_hbm.at[gidx[j]], rows[j], semg)

    def wait_gather(j):
        pltpu.make_async_copy(g_hbm.at[gidx[j]], rows[j], semg).wait()

    def issue_scatter(j):
        pltpu.async_copy(rows[j], acc.at[didx[j]], sems, add=True)

    def wait_scatter(j):
        pltpu.make_async_copy(rows[j], acc.at[didx[j]], sems).wait()

    zero16 = jnp.zeros((16,), jnp.float32)

    def zbody(i, _):
        rows0[i // 8, pl.ds((i % 8) * 16, 16)] = zero16
        return 0
    lax.fori_loop(0, CHUNK * HH // 16, zbody, 0)

    def zcopy(i, _):
        pltpu.sync_copy(rows0, acc.at[pl.ds(s * RPT + i * CHUNK, CHUNK)])
        return 0
    lax.fori_loop(0, RPT // CHUNK, zcopy, 0)
    plsc.subcore_barrier()

    issue_idx(0, 0)
    issue_idx(1, 1)
    wait_idx(0, 0)
    issue_gather(0)

    def quad_body(i, _):
        for b in range(4):
            k = 4 * i + b

            @pl.when(k >= 2)
            def _wait_scatter_k2():
                wait_scatter((b + 2) % 4)

            @pl.when(k + 2 < NCH)
            def _prefetch_idx():
                issue_idx(k + 2, (b + 2) % 4)

            @pl.when(k + 1 < NCH)
            def _issue_next_gather():
                wait_idx(k + 1, (b + 1) % 4)
                issue_gather((b + 1) % 4)

            wait_gather(b)
            issue_scatter(b)
        return 0
    lax.fori_loop(0, NCH // 4, quad_body, 0)
    wait_scatter((NCH - 2) % 4)
    wait_scatter((NCH - 1) % 4)
    plsc.subcore_barrier()

    def obody(i, _):
        r0 = s * RPT + i * CHUNK
        pltpu.sync_copy(acc.at[pl.ds(r0, CHUNK)], rows0)
        pltpu.sync_copy(rows0, out_hbm.at[pl.ds(c * NPAD + r0, CHUNK)])
        return 0
    lax.fori_loop(0, RPT // CHUNK, obody, 0)


@functools.cache
def _agg1_kernel():
    return pl.kernel(
        _agg1_body,
        out_type=jax.ShapeDtypeStruct((NC * NPAD,), jnp.float32),
        mesh=_mesh(),
        compiler_params=pltpu.CompilerParams(needs_layout_passes=False),
        scratch_types=[
            pltpu.VMEM((NPAD,), jnp.float32),
            pltpu.VMEM((EPD,), jnp.int32),
            pltpu.VMEM((EPD,), jnp.int32),
            pltpu.VMEM((NPAD,), jnp.float32),
            pltpu.VMEM((RPT,), jnp.float32),
            pltpu.VMEM((RPT,), jnp.float32),
            pltpu.VMEM_SHARED((NS, NPAD), jnp.float32),
        ],
    )


def _agg1_body(g3_hbm, src_hbm, dst_hbm, out_hbm, g3b, srcb, dstb, acc,
               red, res, shared):
    c = lax.axis_index("c")
    s = lax.axis_index("s")
    tile = c * NS + s
    pltpu.sync_copy(g3_hbm, g3b)
    pltpu.sync_copy(src_hbm.at[pl.ds(tile * EPD, EPD)], srcb)
    pltpu.sync_copy(dst_hbm.at[pl.ds(tile * EPD, EPD)], dstb)

    zero16 = jnp.zeros((16,), jnp.float32)

    def zbody(i, _):
        acc[pl.ds(i * 16, 16)] = zero16
        return 0
    lax.fori_loop(0, NPAD // 16, zbody, 0)

    def ebody(i, _):
        si = srcb[pl.ds(i * 16, 16)]
        di = dstb[pl.ds(i * 16, 16)]
        vals = plsc.load_gather(g3b, [si])
        plsc.addupdate_scatter(acc, [di], vals)
        return 0
    lax.fori_loop(0, EPD // 16, ebody, 0)

    pltpu.sync_copy(acc, shared.at[s])
    plsc.subcore_barrier()

    pltpu.sync_copy(shared.at[0, pl.ds(s * RPT, RPT)], res)

    def rbody(r, _):
        pltpu.sync_copy(shared.at[r, pl.ds(s * RPT, RPT)], red)

        def abody(j, _):
            res[pl.ds(j * 16, 16)] = res[pl.ds(j * 16, 16)] + red[pl.ds(j * 16, 16)]
            return 0
        lax.fori_loop(0, RPT // 16, abody, 0)
        return 0
    lax.fori_loop(1, NS, rbody, 0)

    pltpu.sync_copy(res, out_hbm.at[pl.ds(c * NPAD + s * RPT, RPT)])


def _dinv_of(degp_blk):
    deg = degp_blk[:, 0] + degp_blk[:, 1] + 1.0
    return lax.rsqrt(jnp.maximum(deg, 1.0))


def _tc1_body(x_ref, w1_ref, degp_ref, g1_ref):
    h = jnp.dot(x_ref[...].astype(jnp.bfloat16), w1_ref[...].astype(jnp.bfloat16),
                preferred_element_type=jnp.float32)
    dinv = _dinv_of(degp_ref[...])
    g = h * dinv[:, None]
    g1_ref[0] = g[:, :HH]
    g1_ref[1] = g[:, HH:]


def _tc2_body(agg_ref, g_ref, degp_ref, b_ref, w_ref, out_ref):
    dinv = _dinv_of(degp_ref[...])
    a = jnp.concatenate([agg_ref[0], agg_ref[1]], axis=1)
    g = jnp.concatenate([g_ref[0], g_ref[1]], axis=1)
    h = jnp.maximum(dinv[:, None] * (a + g) + b_ref[0, :], 0.0)
    g2 = jnp.dot(h.astype(jnp.bfloat16), w_ref[...].astype(jnp.bfloat16),
                 preferred_element_type=jnp.float32) * dinv[:, None]
    out_ref[0] = g2[:, :HH]
    out_ref[1] = g2[:, HH:]


def _tc3_body(agg_ref, g_ref, degp_ref, b_ref, w3_ref, out_ref):
    dinv = _dinv_of(degp_ref[...])
    a = jnp.concatenate([agg_ref[0], agg_ref[1]], axis=1)
    g = jnp.concatenate([g_ref[0], g_ref[1]], axis=1)
    h = jnp.maximum(dinv[:, None] * (a + g) + b_ref[0, :], 0.0)
    hb = h.astype(jnp.bfloat16).astype(jnp.float32)
    wb = w3_ref[0, :].astype(jnp.bfloat16).astype(jnp.float32)
    t = jnp.sum(hb * wb[None, :], axis=1) * dinv
    out_ref[...] = jnp.broadcast_to(t[:, None], (BM, 8))


def _tc4_body(p3_ref, g3_ref, degp_ref, b3_ref, out_ref):
    dinv = _dinv_of(degp_ref[...])
    agg = p3_ref[:, 0] + p3_ref[:, 1]
    res = dinv * (agg + g3_ref[:, 0]) + b3_ref[0, 0]
    out_ref[...] = jnp.broadcast_to(res[:, None], (BM, 8))


def _tc1(x, W1, degpT):
    return pl.pallas_call(
        _tc1_body,
        grid=(GRID,),
        in_specs=[
            pl.BlockSpec((BM, D), lambda i: (i, 0)),
            pl.BlockSpec((D, H), lambda i: (0, 0)),
            pl.BlockSpec((BM, 2), lambda i: (i, 0)),
        ],
        out_specs=pl.BlockSpec((2, BM, HH), lambda i: (0, i, 0)),
        out_shape=jax.ShapeDtypeStruct((2, N, HH), jnp.float32),
    )(x, W1, degpT)


def _tc2(agg, g, degpT, b, W):
    return pl.pallas_call(
        _tc2_body,
        grid=(GRID,),
        in_specs=[
            pl.BlockSpec((2, BM, HH), lambda i: (0, i, 0)),
            pl.BlockSpec((2, BM, HH), lambda i: (0, i, 0)),
            pl.BlockSpec((BM, 2), lambda i: (i, 0)),
            pl.BlockSpec((1, H), lambda i: (0, 0)),
            pl.BlockSpec((H, H), lambda i: (0, 0)),
        ],
        out_specs=pl.BlockSpec((2, BM, HH), lambda i: (0, i, 0)),
        out_shape=jax.ShapeDtypeStruct((2, N, HH), jnp.float32),
    )(agg, g, degpT, b, W)


def _tc3(agg, g, degpT, b, W3r):
    return pl.pallas_call(
        _tc3_body,
        grid=(GRID,),
        in_specs=[
            pl.BlockSpec((2, BM, HH), lambda i: (0, i, 0)),
            pl.BlockSpec((2, BM, HH), lambda i: (0, i, 0)),
            pl.BlockSpec((BM, 2), lambda i: (i, 0)),
            pl.BlockSpec((1, H), lambda i: (0, 0)),
            pl.BlockSpec((1, H), lambda i: (0, 0)),
        ],
        out_specs=pl.BlockSpec((BM, 8), lambda i: (i, 0)),
        out_shape=jax.ShapeDtypeStruct((N, 8), jnp.float32),
    )(agg, g, degpT, b, W3r)


def _tc4(p3T, g3r, degpT, b3):
    return pl.pallas_call(
        _tc4_body,
        grid=(GRID,),
        in_specs=[
            pl.BlockSpec((BM, 2), lambda i: (i, 0)),
            pl.BlockSpec((BM, 8), lambda i: (i, 0)),
            pl.BlockSpec((BM, 2), lambda i: (i, 0)),
            pl.BlockSpec((1, 1), lambda i: (0, 0)),
        ],
        out_specs=pl.BlockSpec((BM, 8), lambda i: (i, 0)),
        out_shape=jax.ShapeDtypeStruct((N, 8), jnp.float32),
    )(p3T, g3r, degpT, b3)


@jax.jit
def _run(x, src, dst, W1, b1, W2, b2, W3, b3):
    pad = EPAD - E
    srcp = jnp.concatenate([src, jnp.zeros((pad,), jnp.int32)])
    srcpp = jnp.concatenate([srcp, srcp + N])
    dstp = jnp.concatenate([dst, jnp.full((pad,), N, jnp.int32)])

    degpT = _deg_kernel()(dst).reshape(NC, NPAD).T
    g1 = _tc1(x, W1, degpT)
    agg1 = _agg_kernel()(g1.reshape(NC * N, HH), srcpp, dstp).reshape(NC, NPAD, HH)
    g2 = _tc2(agg1, g1, degpT, b1.reshape(1, H), W2)
    agg2 = _agg_kernel()(g2.reshape(NC * N, HH), srcpp, dstp).reshape(NC, NPAD, HH)
    g3 = _tc3(agg2, g2, degpT, b2.reshape(1, H), W3.reshape(1, H))
    g3p = jnp.concatenate([g3[:, 0], jnp.zeros((NPAD - N,), jnp.float32)])
    p3T = _agg1_kernel()(g3p, src, dst).reshape(NC, NPAD).T
    out = _tc4(p3T, g3, degpT, b3.reshape(1, 1))
    return out[:, :1]


def kernel(x, edge_index, edge_attr, W1, b1, W2, b2, W3, b3):
    src = edge_index[0].astype(jnp.int32)
    dst = edge_index[1].astype(jnp.int32)
    return _run(x, src, dst, W1, b1, W2, b2, W3, b3)
